# baseline (speedup 1.0000x reference)
import jax
import jax.numpy as jnp
from jax.experimental import pallas as pl
from jax.experimental.pallas import tpu as pltpu

_KB = 1024
_WB = 1024


def _body(x_ref, a_ref, p_ref, out_ref, xtf_ref, acc_ref):
    w = pl.program_id(0)
    v = pl.program_id(1)
    nv = pl.num_programs(1)
    nbc = x_ref.shape[0]
    ll = x_ref.shape[2]
    half = acc_ref.shape[0] // 2

    @pl.when(w == 0)
    def _build_chunk():
        xc = x_ref[...]
        pieces = []
        for c in range(2):
            for bb in range(nbc // 2):
                pieces.append(xc[bb * 2 + c].T)
        xtf_ref[v] = jnp.concatenate(pieces, axis=0)

    lhs = xtf_ref[v]
    impc = jnp.where(jnp.isneginf(lhs), 0.0, lhs).astype(jnp.bfloat16)
    contrib = jnp.dot(
        impc,
        a_ref[...].astype(jnp.bfloat16),
        preferred_element_type=jnp.float32,
    )

    @pl.when(v == 0)
    def _init():
        acc_ref[...] = contrib

    @pl.when(v != 0)
    def _acc():
        acc_ref[...] += contrib

    @pl.when(v == nv - 1)
    def _epilogue():
        xt = xtf_ref[w]
        mask = jnp.isneginf(xt)
        imp = jnp.where(mask, 0.0, xt)
        acc = acc_ref[...]
        imp0, imp1 = imp[:half], imp[half:]
        x10, x11 = acc[:half], acc[half:]
        g0 = (p_ref[0, 0] * imp0 + p_ref[0, 1] * imp1
              + p_ref[0, 2] * x10 + p_ref[0, 3] * x11 + p_ref[0, 4])
        g1 = (p_ref[1, 0] * imp0 + p_ref[1, 1] * imp1
              + p_ref[1, 2] * x10 + p_ref[1, 3] * x11 + p_ref[1, 4])
        gcn = jnp.concatenate([g0, g1], axis=0)
        ot = jnp.where(mask, gcn, imp)
        pieces = []
        for bb in range(nbc // 2):
            for c in range(2):
                r0 = c * half + bb * ll
                pieces.append(ot[r0:r0 + ll, :].T)
        out_ref[...] = jnp.stack(pieces, axis=0)


def kernel(x, supports, W, b):
    B, C, N, L = x.shape
    R = C * B * L
    a = supports[0]
    xr = x.reshape(B * C, N, L)
    params = jnp.concatenate([W, b[:, None]], axis=1)
    nv = N // _KB

    out_n = pl.pallas_call(
        _body,
        grid=(N // _WB, nv),
        in_specs=[
            pl.BlockSpec(
                (B * C, _KB, L),
                lambda w, v, _nv=nv: (0, jnp.where(w == 0, v, _nv - 1), 0),
            ),
            pl.BlockSpec((_KB, _WB), lambda w, v: (v, w)),
            pl.BlockSpec(memory_space=pltpu.SMEM),
        ],
        out_specs=pl.BlockSpec((B * C, _WB, L), lambda w, v: (0, w, 0)),
        out_shape=jax.ShapeDtypeStruct((B * C, N, L), jnp.float32),
        scratch_shapes=[
            pltpu.VMEM((nv, R, _KB), jnp.float32),
            pltpu.VMEM((R, _WB), jnp.float32),
        ],
    )(xr, a, params)

    return out_n.reshape(B, C, N, L)


# speedup vs baseline: 1.4169x; 1.4169x over previous
import jax
import jax.numpy as jnp
from jax.experimental import pallas as pl
from jax.experimental.pallas import tpu as pltpu

_VB = 512


def _body(xc_ref, a_ref, xt_ref, p_ref, out_ref):
    v = pl.program_id(0)
    nv = pl.num_programs(0)

    xc = xc_ref[...]
    impc = jnp.where(jnp.isneginf(xc), 0.0, xc).astype(jnp.bfloat16)
    contrib = jnp.dot(
        impc,
        a_ref[...].astype(jnp.bfloat16),
        preferred_element_type=jnp.float32,
    )

    @pl.when(v == 0)
    def _init():
        out_ref[...] = contrib

    @pl.when(v != 0)
    def _acc():
        out_ref[...] += contrib

    @pl.when(v == nv - 1)
    def _epilogue():
        xt = xt_ref[...]
        mask = jnp.isneginf(xt)
        imp = jnp.where(mask, 0.0, xt)
        acc = out_ref[...]
        half = imp.shape[0] // 2
        imp0, imp1 = imp[:half], imp[half:]
        x10, x11 = acc[:half], acc[half:]
        g0 = (p_ref[0, 0] * imp0 + p_ref[0, 1] * imp1
              + p_ref[0, 2] * x10 + p_ref[0, 3] * x11 + p_ref[0, 4])
        g1 = (p_ref[1, 0] * imp0 + p_ref[1, 1] * imp1
              + p_ref[1, 2] * x10 + p_ref[1, 3] * x11 + p_ref[1, 4])
        gcn = jnp.concatenate([g0, g1], axis=0)
        out_ref[...] = jnp.where(mask, gcn, imp)


def kernel(x, supports, W, b):
    B, C, N, L = x.shape
    R = C * B * L
    a = supports[0]
    xt = jnp.transpose(x, (1, 0, 3, 2)).reshape(R, N)
    params = jnp.concatenate([W, b[:, None]], axis=1)

    out_t = pl.pallas_call(
        _body,
        grid=(N // _VB,),
        in_specs=[
            pl.BlockSpec((R, _VB), lambda v: (0, v)),
            pl.BlockSpec((_VB, N), lambda v: (v, 0)),
            pl.BlockSpec((R, N), lambda v: (0, 0)),
            pl.BlockSpec(memory_space=pltpu.SMEM),
        ],
        out_specs=pl.BlockSpec((R, N), lambda v: (0, 0)),
        out_shape=jax.ShapeDtypeStruct((R, N), jnp.float32),
    )(xt, a, xt, params)

    return out_t.reshape(C, B, L, N).transpose(1, 0, 3, 2)
